# baseline (speedup 1.0000x reference)
import functools

import jax
import jax.numpy as jnp
from jax import lax
from jax.experimental import pallas as pl
from jax.experimental.pallas import tpu as pltpu
from jax.experimental.pallas import tpu_sc as plsc

BATCH = 4096
SEQ = 200
DIM = 64
PDIM = 128
NC = 2
NS = 16
NW = NC * NS
IDX_PER_W = BATCH * SEQ // NW
CHUNK = 100
NCHUNK = IDX_PER_W // CHUNK
NBUF = 4
NGRP = NCHUNK // NBUF


def _sc_embed(x_idx, tok_pad, pos_table):
    mesh = plsc.VectorSubcoreMesh(core_axis_name="c", subcore_axis_name="s")

    @functools.partial(
        pl.kernel,
        mesh=mesh,
        out_type=jax.ShapeDtypeStruct((BATCH * SEQ, PDIM), jnp.float32),
        compiler_params=pltpu.CompilerParams(use_tc_tiling_on_sc=False),
        scratch_types=[
            pltpu.VMEM((NCHUNK, CHUNK), jnp.int32),
            pltpu.VMEM((SEQ, DIM), jnp.float32),
            pltpu.VMEM((CHUNK, PDIM), jnp.float32),
            pltpu.VMEM((CHUNK, PDIM), jnp.float32),
            pltpu.VMEM((CHUNK, PDIM), jnp.float32),
            pltpu.VMEM((CHUNK, PDIM), jnp.float32),
            pltpu.SemaphoreType.DMA,
            pltpu.SemaphoreType.DMA,
            pltpu.SemaphoreType.DMA,
            pltpu.SemaphoreType.DMA,
            pltpu.SemaphoreType.DMA,
            pltpu.SemaphoreType.DMA,
            pltpu.SemaphoreType.DMA,
            pltpu.SemaphoreType.DMA,
        ],
    )
    def k(x_hbm, tok_hbm, pos_hbm, out_hbm, idx_v, pos_v,
          b0, b1, b2, b3, si0, si1, si2, si3, so0, so1, so2, so3):
        bufs = [b0, b1, b2, b3]
        isems = [si0, si1, si2, si3]
        osems = [so0, so1, so2, so3]
        wid = lax.axis_index("s") * NC + lax.axis_index("c")
        row0 = wid * IDX_PER_W
        pltpu.sync_copy(x_hbm.at[wid], idx_v)
        pltpu.sync_copy(pos_hbm, pos_v)

        def start_in(j, b):
            pltpu.async_copy(
                tok_hbm.at[idx_v.at[j]], bufs[b], isems[b])

        def wait_in(b):
            pltpu.make_async_copy(
                tok_hbm.at[pl.ds(0, CHUNK)], bufs[b], isems[b]).wait()

        def start_out(j, b):
            pltpu.async_copy(
                bufs[b].at[:, pl.ds(0, DIM)],
                out_hbm.at[pl.ds(row0 + j * CHUNK, CHUNK), pl.ds(0, DIM)],
                osems[b])

        def wait_out(b):
            pltpu.make_async_copy(
                bufs[b].at[:, pl.ds(0, DIM)],
                out_hbm.at[pl.ds(0, CHUNK), pl.ds(0, DIM)], osems[b]).wait()

        def add_pos(b):
            buf = bufs[b]
            pbase = (b % 2) * CHUNK

            def add_body(r, c2):
                for rr in range(2):
                    for jj in range(DIM // 16):
                        plsc.addupdate(
                            buf.at[2 * r + rr, pl.ds(jj * 16, 16)],
                            pos_v[pbase + 2 * r + rr, pl.ds(jj * 16, 16)])
                return c2

            lax.fori_loop(0, CHUNK // 2, add_body, 0)

        def step(j, b, first, last):
            wait_in(b)
            add_pos(b)
            start_out(j, b)
            nb = (b + 2) % NBUF
            if not first:
                wait_out(nb)
            if not last:
                start_in(j + 2, nb)

        start_in(0, 0)
        start_in(1, 1)

        step(0, 0, True, False)
        step(1, 1, True, False)
        step(2, 2, False, False)
        step(3, 3, False, False)

        def grp(g, carry):
            j = g * NBUF
            for b in range(NBUF):
                step(j + b, b, False, False)
            return carry

        lax.fori_loop(1, NGRP - 1, grp, 0)

        j = (NGRP - 1) * NBUF
        step(j + 0, 0, False, False)
        step(j + 1, 1, False, False)
        step(j + 2, 2, False, True)
        step(j + 3, 3, False, True)

        wait_out(2)
        wait_out(3)

    return k(x_idx, tok_pad, pos_table)


def kernel(x, token_table, pos_table):
    x_idx = x.astype(jnp.int32).reshape(NW, NCHUNK, CHUNK)
    tok_pad = jnp.pad(token_table, ((0, 0), (0, PDIM - DIM)))
    out_pad = _sc_embed(x_idx, tok_pad, pos_table)
    return out_pad.reshape(BATCH, SEQ, PDIM)[:, :, :DIM]


# speedup vs baseline: 1.1218x; 1.1218x over previous
import functools

import jax
import jax.numpy as jnp
from jax import lax
from jax.experimental import pallas as pl
from jax.experimental.pallas import tpu as pltpu
from jax.experimental.pallas import tpu_sc as plsc

BATCH = 4096
SEQ = 200
DIM = 64
PDIM = 128
NC = 2
NS = 16
NW = NC * NS
IDX_PER_W = BATCH * SEQ // NW
CHUNK = 100
NCHUNK = IDX_PER_W // CHUNK
NBUF = 4
NGRP = NCHUNK // NBUF


def _sc_embed(x_idx, tok_pad, pos_table):
    mesh = plsc.VectorSubcoreMesh(core_axis_name="c", subcore_axis_name="s")

    @functools.partial(
        pl.kernel,
        mesh=mesh,
        out_type=jax.ShapeDtypeStruct((BATCH * SEQ, PDIM), jnp.float32),
        compiler_params=pltpu.CompilerParams(use_tc_tiling_on_sc=False),
        scratch_types=[
            pltpu.VMEM((NCHUNK, CHUNK), jnp.int32),
            pltpu.VMEM((SEQ, DIM), jnp.float32),
            pltpu.VMEM((CHUNK, PDIM), jnp.float32),
            pltpu.VMEM((CHUNK, PDIM), jnp.float32),
            pltpu.VMEM((CHUNK, PDIM), jnp.float32),
            pltpu.VMEM((CHUNK, PDIM), jnp.float32),
            pltpu.SemaphoreType.DMA,
            pltpu.SemaphoreType.DMA,
            pltpu.SemaphoreType.DMA,
            pltpu.SemaphoreType.DMA,
            pltpu.SemaphoreType.DMA,
            pltpu.SemaphoreType.DMA,
            pltpu.SemaphoreType.DMA,
            pltpu.SemaphoreType.DMA,
        ],
    )
    def k(x_hbm, tok_hbm, pos_hbm, out_hbm, idx_v, pos_v,
          b0, b1, b2, b3, si0, si1, si2, si3, so0, so1, so2, so3):
        bufs = [b0, b1, b2, b3]
        isems = [si0, si1, si2, si3]
        osems = [so0, so1, so2, so3]
        wid = lax.axis_index("s") * NC + lax.axis_index("c")
        row0 = wid * IDX_PER_W
        pltpu.sync_copy(x_hbm.at[wid], idx_v)
        pltpu.sync_copy(pos_hbm, pos_v)

        def start_in(j, b):
            pltpu.async_copy(
                tok_hbm.at[idx_v.at[j]], bufs[b], isems[b])

        def wait_in(b):
            pltpu.make_async_copy(
                tok_hbm.at[pl.ds(0, CHUNK)], bufs[b], isems[b]).wait()

        def start_out(j, b):
            pltpu.async_copy(
                bufs[b], out_hbm.at[pl.ds(row0 + j * CHUNK, CHUNK)],
                osems[b])

        def wait_out(b):
            pltpu.make_async_copy(
                bufs[b], out_hbm.at[pl.ds(0, CHUNK)], osems[b]).wait()

        def add_pos(b):
            buf = bufs[b]
            pbase = (b % 2) * CHUNK

            def add_body(r, c2):
                for rr in range(2):
                    for jj in range(DIM // 16):
                        plsc.addupdate(
                            buf.at[2 * r + rr, pl.ds(jj * 16, 16)],
                            pos_v[pbase + 2 * r + rr, pl.ds(jj * 16, 16)])
                return c2

            lax.fori_loop(0, CHUNK // 2, add_body, 0)

        def step(j, b, first, last):
            wait_in(b)
            nb = (b + 2) % NBUF
            if not first:
                wait_out(nb)
            if not last:
                start_in(j + 2, nb)
            add_pos(b)
            start_out(j, b)

        start_in(0, 0)
        start_in(1, 1)

        step(0, 0, True, False)
        step(1, 1, True, False)
        step(2, 2, False, False)
        step(3, 3, False, False)

        def grp(g, carry):
            j = g * NBUF
            for b in range(NBUF):
                step(j + b, b, False, False)
            return carry

        lax.fori_loop(1, NGRP - 1, grp, 0)

        j = (NGRP - 1) * NBUF
        step(j + 0, 0, False, False)
        step(j + 1, 1, False, False)
        step(j + 2, 2, False, True)
        step(j + 3, 3, False, True)

        wait_out(2)
        wait_out(3)

    return k(x_idx, tok_pad, pos_table)


def kernel(x, token_table, pos_table):
    x_idx = x.astype(jnp.int32).reshape(NW, NCHUNK, CHUNK)
    tok_pad = jnp.pad(token_table, ((0, 0), (0, PDIM - DIM)))
    out_pad = _sc_embed(x_idx, tok_pad, pos_table)
    return out_pad.reshape(BATCH, SEQ, PDIM)[:, :, :DIM]


# speedup vs baseline: 1.1251x; 1.0030x over previous
import functools

import jax
import jax.numpy as jnp
from jax import lax
from jax.experimental import pallas as pl
from jax.experimental.pallas import tpu as pltpu
from jax.experimental.pallas import tpu_sc as plsc

BATCH = 4096
SEQ = 200
DIM = 64
PDIM = 128
NC = 2
NS = 16
NW = NC * NS
IDX_PER_W = BATCH * SEQ // NW
CHUNK = 100
NCHUNK = IDX_PER_W // CHUNK
NBUF = 4
NGRP = NCHUNK // NBUF


def _sc_embed(x_idx, tok_pad, pos_table):
    mesh = plsc.VectorSubcoreMesh(core_axis_name="c", subcore_axis_name="s")

    @functools.partial(
        pl.kernel,
        mesh=mesh,
        out_type=jax.ShapeDtypeStruct((BATCH * SEQ, PDIM), jnp.float32),
        compiler_params=pltpu.CompilerParams(use_tc_tiling_on_sc=False),
        scratch_types=[
            pltpu.VMEM((NCHUNK, CHUNK), jnp.int32),
            pltpu.VMEM((SEQ, DIM), jnp.float32),
            pltpu.VMEM((CHUNK, PDIM), jnp.float32),
            pltpu.VMEM((CHUNK, PDIM), jnp.float32),
            pltpu.VMEM((CHUNK, PDIM), jnp.float32),
            pltpu.VMEM((CHUNK, PDIM), jnp.float32),
            pltpu.SemaphoreType.DMA,
            pltpu.SemaphoreType.DMA,
            pltpu.SemaphoreType.DMA,
            pltpu.SemaphoreType.DMA,
            pltpu.SemaphoreType.DMA,
            pltpu.SemaphoreType.DMA,
            pltpu.SemaphoreType.DMA,
            pltpu.SemaphoreType.DMA,
        ],
    )
    def k(x_hbm, tok_hbm, pos_hbm, out_hbm, idx_v, pos_v,
          b0, b1, b2, b3, si0, si1, si2, si3, so0, so1, so2, so3):
        bufs = [b0, b1, b2, b3]
        isems = [si0, si1, si2, si3]
        osems = [so0, so1, so2, so3]
        wid = lax.axis_index("s") * NC + lax.axis_index("c")
        row0 = wid * IDX_PER_W
        pltpu.sync_copy(x_hbm.at[wid], idx_v)
        pltpu.sync_copy(pos_hbm, pos_v)

        def start_in(j, b):
            pltpu.async_copy(
                tok_hbm.at[idx_v.at[j]], bufs[b], isems[b])

        def wait_in(b):
            pltpu.make_async_copy(
                tok_hbm.at[pl.ds(0, CHUNK)], bufs[b], isems[b]).wait()

        def start_out(j, b):
            pltpu.async_copy(
                bufs[b], out_hbm.at[pl.ds(row0 + j * CHUNK, CHUNK)],
                osems[b])

        def wait_out(b):
            pltpu.make_async_copy(
                bufs[b], out_hbm.at[pl.ds(0, CHUNK)], osems[b]).wait()

        def add_pos(b):
            buf = bufs[b]
            pbase = (b % 2) * CHUNK

            def add_body(r, c2):
                for rr in range(4):
                    for jj in range(DIM // 16):
                        plsc.addupdate(
                            buf.at[4 * r + rr, pl.ds(jj * 16, 16)],
                            pos_v[pbase + 4 * r + rr, pl.ds(jj * 16, 16)])
                return c2

            lax.fori_loop(0, CHUNK // 4, add_body, 0)

        def step(j, b, first, last):
            wait_in(b)
            nb = (b + 2) % NBUF
            if not first:
                wait_out(nb)
            if not last:
                start_in(j + 2, nb)
            add_pos(b)
            start_out(j, b)

        start_in(0, 0)
        start_in(1, 1)

        step(0, 0, True, False)
        step(1, 1, True, False)
        step(2, 2, False, False)
        step(3, 3, False, False)

        def grp(g, carry):
            j = g * NBUF
            for b in range(NBUF):
                step(j + b, b, False, False)
            return carry

        lax.fori_loop(1, NGRP - 1, grp, 0)

        j = (NGRP - 1) * NBUF
        step(j + 0, 0, False, False)
        step(j + 1, 1, False, False)
        step(j + 2, 2, False, True)
        step(j + 3, 3, False, True)

        wait_out(2)
        wait_out(3)

    return k(x_idx, tok_pad, pos_table)


def kernel(x, token_table, pos_table):
    x_idx = x.astype(jnp.int32).reshape(NW, NCHUNK, CHUNK)
    tok_pad = jnp.pad(token_table, ((0, 0), (0, PDIM - DIM)))
    out_pad = _sc_embed(x_idx, tok_pad, pos_table)
    return out_pad.reshape(BATCH, SEQ, PDIM)[:, :, :DIM]
